# baseline (speedup 1.0000x reference)
import jax
import jax.numpy as jnp
from jax import lax
from jax.experimental import pallas as pl
from jax.experimental.pallas import tpu as pltpu

DEPTH_CONST = 1000
BATCH_BLOCK = 64


def _body(onoff_ref, x_ref, out_ref):
    bb, s = x_ref.shape
    cols = lax.broadcasted_iota(jnp.int32, (bb, s, DEPTH_CONST), 2)
    oh = cols == x_ref[...][:, :, None]
    out_ref[...] = jnp.where(oh, onoff_ref[0, 0], onoff_ref[0, 1])


def kernel(x, on_value, off_value):
    B, S = x.shape
    g = B // BATCH_BLOCK
    onoff = jnp.stack([on_value, off_value]).reshape(1, 2)
    out = pl.pallas_call(
        _body,
        grid=(g,),
        in_specs=[
            pl.BlockSpec(memory_space=pltpu.SMEM),
            pl.BlockSpec((BATCH_BLOCK, S), lambda i: (i, 0)),
        ],
        out_specs=pl.BlockSpec((BATCH_BLOCK, S, DEPTH_CONST), lambda i: (i, 0, 0)),
        out_shape=jax.ShapeDtypeStruct((B, S, DEPTH_CONST), jnp.float32),
    )(onoff, x)
    return out


# speedup vs baseline: 1.0042x; 1.0042x over previous
import jax
import jax.numpy as jnp
from jax import lax
from jax.experimental import pallas as pl
from jax.experimental.pallas import tpu as pltpu

DEPTH_CONST = 1000
BATCH_BLOCK = 16
NBUF = 8


def _body(onoff_ref, x_ref, out_ref, *scratch):
    bufs = scratch[:NBUF]
    sems = scratch[NBUF:]
    n_chunks = x_ref.shape[0] // BATCH_BLOCK
    n_outer = n_chunks // NBUF
    cols = lax.broadcasted_iota(
        jnp.int32, (BATCH_BLOCK, x_ref.shape[1], DEPTH_CONST), 2
    )
    on = onoff_ref[0, 0]
    off = onoff_ref[0, 1]

    def outer(i, carry):
        for b in range(NBUF):
            chunk = i * NBUF + b

            @pl.when(i > 0)
            def _wait():
                pltpu.make_async_copy(
                    bufs[b],
                    out_ref.at[pl.ds(chunk * BATCH_BLOCK, BATCH_BLOCK)],
                    sems[b],
                ).wait()

            idx = x_ref[pl.ds(chunk * BATCH_BLOCK, BATCH_BLOCK), :]
            oh = cols == idx[:, :, None]
            bufs[b][...] = jnp.where(oh, on, off)
            pltpu.make_async_copy(
                bufs[b],
                out_ref.at[pl.ds(chunk * BATCH_BLOCK, BATCH_BLOCK)],
                sems[b],
            ).start()
        return carry

    lax.fori_loop(0, n_outer, outer, 0)
    for b in range(NBUF):
        pltpu.make_async_copy(
            bufs[b],
            out_ref.at[pl.ds(0, BATCH_BLOCK)],
            sems[b],
        ).wait()


def kernel(x, on_value, off_value):
    B, S = x.shape
    onoff = jnp.stack([on_value, off_value]).reshape(1, 2)
    out = pl.pallas_call(
        _body,
        in_specs=[
            pl.BlockSpec(memory_space=pltpu.SMEM),
            pl.BlockSpec(memory_space=pltpu.VMEM),
        ],
        out_specs=pl.BlockSpec(memory_space=pl.ANY),
        out_shape=jax.ShapeDtypeStruct((B, S, DEPTH_CONST), jnp.float32),
        scratch_shapes=(
            [pltpu.VMEM((BATCH_BLOCK, S, DEPTH_CONST), jnp.float32)] * NBUF
            + [pltpu.SemaphoreType.DMA] * NBUF
        ),
    )(onoff, x)
    return out


# speedup vs baseline: 4.4411x; 4.4225x over previous
import jax
import jax.numpy as jnp
from jax import lax
from jax.experimental import pallas as pl
from jax.experimental.pallas import tpu as pltpu

DEPTH_CONST = 1000
NBUF = 5


def _body(onoff_ref, xt_ref, out_ref, *scratch):
    bufs = scratch[:NBUF]
    sems = scratch[NBUF:]
    s, b = xt_ref.shape
    n_outer = s // NBUF
    rows = lax.broadcasted_iota(jnp.int32, (DEPTH_CONST, b), 0)
    on = onoff_ref[0, 0]
    off = onoff_ref[0, 1]

    def outer(i, carry):
        for k in range(NBUF):
            j = i * NBUF + k

            @pl.when(i > 0)
            def _wait():
                pltpu.make_async_copy(
                    bufs[k], out_ref.at[pl.ds(j, 1)], sems[k]
                ).wait()

            xj = xt_ref[pl.ds(j, 1), :]
            oh = rows == jnp.broadcast_to(xj, (DEPTH_CONST, b))
            bufs[k][0] = jnp.where(oh, on, off)
            pltpu.make_async_copy(
                bufs[k], out_ref.at[pl.ds(j, 1)], sems[k]
            ).start()
        return carry

    lax.fori_loop(0, n_outer, outer, 0)
    for k in range(NBUF):
        pltpu.make_async_copy(bufs[k], out_ref.at[pl.ds(0, 1)], sems[k]).wait()


def kernel(x, on_value, off_value):
    B, S = x.shape
    onoff = jnp.stack([on_value, off_value]).reshape(1, 2)
    xt = x.T
    out = pl.pallas_call(
        _body,
        in_specs=[
            pl.BlockSpec(memory_space=pltpu.SMEM),
            pl.BlockSpec(memory_space=pltpu.VMEM),
        ],
        out_specs=pl.BlockSpec(memory_space=pl.ANY),
        out_shape=jax.ShapeDtypeStruct((S, DEPTH_CONST, B), jnp.float32),
        scratch_shapes=(
            [pltpu.VMEM((1, DEPTH_CONST, B), jnp.float32)] * NBUF
            + [pltpu.SemaphoreType.DMA] * NBUF
        ),
    )(onoff, xt)
    return out.transpose(2, 0, 1)


# speedup vs baseline: 4.4430x; 1.0004x over previous
import jax
import jax.numpy as jnp
from jax import lax
from jax.experimental import pallas as pl
from jax.experimental.pallas import tpu as pltpu

DEPTH_CONST = 1000
NBUF = 10


def _body(onoff_ref, xt_ref, out_ref, *scratch):
    bufs = scratch[:NBUF]
    sems = scratch[NBUF:]
    s, b = xt_ref.shape
    n_outer = s // NBUF
    rows = lax.broadcasted_iota(jnp.int32, (DEPTH_CONST, b), 0)
    on = onoff_ref[0, 0]
    off = onoff_ref[0, 1]

    def outer(i, carry):
        for k in range(NBUF):
            j = i * NBUF + k

            @pl.when(i > 0)
            def _wait():
                pltpu.make_async_copy(
                    bufs[k], out_ref.at[pl.ds(j, 1)], sems[k]
                ).wait()

            xj = xt_ref[pl.ds(j, 1), :]
            oh = rows == jnp.broadcast_to(xj, (DEPTH_CONST, b))
            bufs[k][0] = jnp.where(oh, on, off)
            pltpu.make_async_copy(
                bufs[k], out_ref.at[pl.ds(j, 1)], sems[k]
            ).start()
        return carry

    lax.fori_loop(0, n_outer, outer, 0)
    for k in range(NBUF):
        pltpu.make_async_copy(bufs[k], out_ref.at[pl.ds(0, 1)], sems[k]).wait()


def kernel(x, on_value, off_value):
    B, S = x.shape
    onoff = jnp.stack([on_value, off_value]).reshape(1, 2)
    xt = x.T
    out = pl.pallas_call(
        _body,
        in_specs=[
            pl.BlockSpec(memory_space=pltpu.SMEM),
            pl.BlockSpec(memory_space=pltpu.VMEM),
        ],
        out_specs=pl.BlockSpec(memory_space=pl.ANY),
        out_shape=jax.ShapeDtypeStruct((S, DEPTH_CONST, B), jnp.float32),
        scratch_shapes=(
            [pltpu.VMEM((1, DEPTH_CONST, B), jnp.float32)] * NBUF
            + [pltpu.SemaphoreType.DMA] * NBUF
        ),
    )(onoff, xt)
    return out.transpose(2, 0, 1)
